# baseline (speedup 1.0000x reference)
import jax
import jax.numpy as jnp
from jax.experimental import pallas as pl
from jax.experimental.pallas import tpu as pltpu

K_TOP = 2


def _router_kernel(x_ref, gw_ref, w_ref, xb_ref):
    x = x_ref[...]
    logits = jnp.dot(x, gw_ref[...].T,
                     preferred_element_type=jnp.float32)
    m = jnp.max(logits, axis=-1, keepdims=True)
    ex = jnp.exp(logits - m)
    p = ex / jnp.sum(ex, axis=-1, keepdims=True)
    num_experts = p.shape[-1]
    idx = jax.lax.broadcasted_iota(jnp.int32, p.shape, 1)
    m1 = jnp.max(p, axis=-1, keepdims=True)
    i1 = jnp.min(jnp.where(p == m1, idx, num_experts), axis=-1, keepdims=True)
    is1 = idx == i1
    p2 = jnp.where(is1, -jnp.inf, p)
    m2 = jnp.max(p2, axis=-1, keepdims=True)
    i2 = jnp.min(jnp.where(p2 == m2, idx, num_experts), axis=-1, keepdims=True)
    sel = is1 | (idx == i2)
    w = jnp.where(sel, p, 0.0)
    w_ref[...] = w / jnp.sum(w, axis=-1, keepdims=True)
    xb_ref[...] = x.astype(jnp.bfloat16)


def _moe_kernel(xb_ref, w_ref, wg_ref, wu_ref, wd_ref, o_ref, h_ref):
    e = pl.program_id(0)
    i = pl.program_id(1)
    n_e = pl.num_programs(0) - 1
    F = wg_ref.shape[-1]
    tb = xb_ref.shape[0]

    @pl.when(e < n_e)
    def _expert():
        xb = xb_ref[...]
        w_all = w_ref[...]
        lane = jax.lax.broadcasted_iota(jnp.int32, w_all.shape, 1)
        we = jnp.sum(jnp.where(lane == e, w_all, 0.0),
                     axis=1, keepdims=True)
        wg = wg_ref[0].astype(jnp.bfloat16)
        wu = wu_ref[0].astype(jnp.bfloat16)
        g = jnp.dot(xb, wg, preferred_element_type=jnp.float32)
        u = jnp.dot(xb, wu, preferred_element_type=jnp.float32)
        h = (g * jax.lax.logistic(g)) * u
        h_ref[pl.ds(i * tb, tb), pl.ds(e * F, F)] = (h * we).astype(jnp.bfloat16)

    @pl.when(e == n_e)
    def _down():
        o_ref[...] = jnp.dot(h_ref[pl.ds(i * tb, tb), :], wd_ref[...],
                             preferred_element_type=jnp.float32)


def kernel(hidden_states, gate_w, w_gate, w_up, w_down,
           mlp_buffer=None, gathered_experts_out_buf=None):
    T, D = hidden_states.shape[0], hidden_states.shape[-1]
    E = gate_w.shape[0]
    F = w_gate.shape[-1]
    x = hidden_states.reshape(T, D)
    wd_all = w_down.reshape(E * F, D).astype(jnp.bfloat16)

    w_dense, xb = pl.pallas_call(
        _router_kernel,
        in_specs=[
            pl.BlockSpec((T, D), lambda: (0, 0)),
            pl.BlockSpec((E, D), lambda: (0, 0)),
        ],
        out_specs=[
            pl.BlockSpec((T, E), lambda: (0, 0)),
            pl.BlockSpec((T, D), lambda: (0, 0)),
        ],
        out_shape=[
            jax.ShapeDtypeStruct((T, E), jnp.float32),
            jax.ShapeDtypeStruct((T, D), jnp.bfloat16),
        ],
    )(x, gate_w)

    clamp = E - 1
    n_tb = 2
    tb = T // n_tb
    out = pl.pallas_call(
        _moe_kernel,
        grid=(E + 1, n_tb),
        in_specs=[
            pl.BlockSpec((tb, D), lambda e, i: (i, 0)),
            pl.BlockSpec((tb, E), lambda e, i: (i, 0)),
            pl.BlockSpec((1, D, F),
                         lambda e, i: (jnp.minimum(e, clamp), 0, 0)),
            pl.BlockSpec((1, D, F),
                         lambda e, i: (jnp.minimum(e, clamp), 0, 0)),
            pl.BlockSpec((E * F, D), lambda e, i: (0, 0)),
        ],
        out_specs=pl.BlockSpec((tb, D), lambda e, i: (i, 0)),
        out_shape=jax.ShapeDtypeStruct((T, D), jnp.float32),
        scratch_shapes=[pltpu.VMEM((T, E * F), jnp.bfloat16)],
    )(xb, w_dense, w_gate, w_up, wd_all)
    return out.reshape(hidden_states.shape)


# speedup vs baseline: 1.1911x; 1.1911x over previous
import jax
import jax.numpy as jnp
from jax.experimental import pallas as pl
from jax.experimental.pallas import tpu as pltpu

K_TOP = 2


def _moe_kernel(x_ref, xb_ref, gw_ref, wg_ref, wu_ref, wd_ref, o_ref, w_ref):
    e = pl.program_id(0)

    @pl.when(e == 0)
    def _router():
        x = x_ref[...]
        logits = jnp.dot(x, gw_ref[...].T,
                         preferred_element_type=jnp.float32)
        m = jnp.max(logits, axis=-1, keepdims=True)
        ex = jnp.exp(logits - m)
        p = ex / jnp.sum(ex, axis=-1, keepdims=True)
        num_experts = p.shape[-1]
        idx = jax.lax.broadcasted_iota(jnp.int32, p.shape, 1)
        m1 = jnp.max(p, axis=-1, keepdims=True)
        i1 = jnp.min(jnp.where(p == m1, idx, num_experts),
                     axis=-1, keepdims=True)
        is1 = idx == i1
        p2 = jnp.where(is1, -jnp.inf, p)
        m2 = jnp.max(p2, axis=-1, keepdims=True)
        i2 = jnp.min(jnp.where(p2 == m2, idx, num_experts),
                     axis=-1, keepdims=True)
        sel = is1 | (idx == i2)
        w = jnp.where(sel, p, 0.0)
        w_ref[...] = w / jnp.sum(w, axis=-1, keepdims=True)

    xb = xb_ref[...]
    w_all = w_ref[...]
    lane = jax.lax.broadcasted_iota(jnp.int32, w_all.shape, 1)
    we = jnp.sum(jnp.where(lane == e, w_all, 0.0),
                 axis=1, keepdims=True)
    wg = wg_ref[0].astype(jnp.bfloat16)
    wu = wu_ref[0].astype(jnp.bfloat16)
    wd = wd_ref[0].astype(jnp.bfloat16)
    g = jnp.dot(xb, wg, preferred_element_type=jnp.float32)
    u = jnp.dot(xb, wu, preferred_element_type=jnp.float32)
    h = (g * jax.lax.logistic(g)) * u
    hw = (h * we).astype(jnp.bfloat16)
    y = jnp.dot(hw, wd, preferred_element_type=jnp.float32)

    @pl.when(e == 0)
    def _init():
        o_ref[...] = y

    @pl.when(e != 0)
    def _acc():
        o_ref[...] += y


def kernel(hidden_states, gate_w, w_gate, w_up, w_down,
           mlp_buffer=None, gathered_experts_out_buf=None):
    T, D = hidden_states.shape[0], hidden_states.shape[-1]
    E = gate_w.shape[0]
    F = w_gate.shape[-1]
    x = hidden_states.reshape(T, D)
    xb = x.astype(jnp.bfloat16)

    out = pl.pallas_call(
        _moe_kernel,
        grid=(E,),
        in_specs=[
            pl.BlockSpec((T, D), lambda e: (0, 0)),
            pl.BlockSpec((T, D), lambda e: (0, 0)),
            pl.BlockSpec((E, D), lambda e: (0, 0)),
            pl.BlockSpec((1, D, F), lambda e: (e, 0, 0)),
            pl.BlockSpec((1, D, F), lambda e: (e, 0, 0)),
            pl.BlockSpec((1, F, D), lambda e: (e, 0, 0)),
        ],
        out_specs=pl.BlockSpec((T, D), lambda e: (0, 0)),
        out_shape=jax.ShapeDtypeStruct((T, D), jnp.float32),
        scratch_shapes=[pltpu.VMEM((T, E), jnp.float32)],
    )(x, xb, gate_w, w_gate, w_up, w_down)
    return out.reshape(hidden_states.shape)
